# baseline (speedup 1.0000x reference)
import functools

import jax
import jax.numpy as jnp
from jax import lax
from jax.experimental import pallas as pl
from jax.experimental.pallas import tpu as pltpu
from jax.experimental.pallas import tpu_sc as plsc

BINS = 1000000
D = 32
NC = 2
NS = 16
NW = NC * NS
B = 4096 * 200
PER_W = B // NW
CH = 512
G = 512
NG = CH // G
NCHUNK = PER_W // CH
HALF = NCHUNK // 2
UNROLL = 16


def _build(interpret=False):
  mesh = plsc.VectorSubcoreMesh(core_axis_name="c", subcore_axis_name="s")

  @functools.partial(
      pl.kernel,
      out_type=jax.ShapeDtypeStruct((B, D), jnp.float32),
      mesh=mesh,
      scratch_types=[
          pltpu.VMEM((PER_W,), jnp.float32),
          pltpu.VMEM((PER_W,), jnp.int32),
          pltpu.VMEM((CH, D), jnp.float32),
          pltpu.VMEM((CH, D), jnp.float32),
          pltpu.SemaphoreType.DMA,
          pltpu.SemaphoreType.DMA,
          pltpu.SemaphoreType.DMA,
      ],
      compiler_params=pltpu.CompilerParams(use_tc_tiling_on_sc=False),
      interpret=interpret,
  )
  def table_lookup(x_hbm, table_hbm, out_hbm,
                   x_all, idx_all, rows0, rows1, sem_g, sem_s0, sem_s1):
    wid = lax.axis_index("s") * NC + lax.axis_index("c")
    base = wid * PER_W

    pltpu.sync_copy(x_hbm.at[pl.ds(base, PER_W)], x_all)

    def bucketize(t, carry):
      for i in range(UNROLL):
        off = t * (16 * UNROLL) + i * 16
        xv = x_all[pl.ds(off, 16)]
        idx_all[pl.ds(off, 16)] = jnp.minimum(
            (xv * float(BINS)).astype(jnp.int32), BINS - 1)
      return carry

    lax.fori_loop(0, PER_W // (16 * UNROLL), bucketize, 0)

    pltpu.make_async_copy(
        rows0, out_hbm.at[pl.ds(base + (NCHUNK - 2) * CH, CH)], sem_s0).start()
    pltpu.make_async_copy(
        rows1, out_hbm.at[pl.ds(base + (NCHUNK - 1) * CH, CH)], sem_s1).start()

    def pipe(t, carry):
      for b in range(2):
        rows = rows0 if b == 0 else rows1
        sem_s = sem_s0 if b == 0 else sem_s1
        c = 2 * t + b
        off = base + c * CH
        pltpu.make_async_copy(rows, out_hbm.at[pl.ds(off, CH)], sem_s).wait()
        gs = [
            pltpu.make_async_copy(
                table_hbm.at[idx_all.at[pl.ds(c * CH + g * G, G)]],
                rows.at[pl.ds(g * G, G)],
                sem_g,
            )
            for g in range(NG)
        ]
        for cp in gs:
          cp.start()
        for cp in gs:
          cp.wait()
        pltpu.make_async_copy(rows, out_hbm.at[pl.ds(off, CH)], sem_s).start()
      return carry

    lax.fori_loop(0, HALF, pipe, 0)

    pltpu.make_async_copy(rows0, out_hbm.at[pl.ds(base, CH)], sem_s0).wait()
    pltpu.make_async_copy(rows1, out_hbm.at[pl.ds(base, CH)], sem_s1).wait()

  return table_lookup


_lookup = _build()


def kernel(x, table):
  xf = x.reshape(B)
  out = _lookup(xf, table)
  return out.reshape(4096, 200, D)


# speedup vs baseline: 1.0214x; 1.0214x over previous
import functools

import jax
import jax.numpy as jnp
from jax import lax
from jax.experimental import pallas as pl
from jax.experimental.pallas import tpu as pltpu
from jax.experimental.pallas import tpu_sc as plsc

BINS = 1000000
D = 32
NC = 2
NS = 16
NW = NC * NS
B = 4096 * 200
PER_W = B // NW
CH = 512
G = 128
NG = CH // G
NCHUNK = PER_W // CH


def _build(interpret=False):
  mesh = plsc.VectorSubcoreMesh(core_axis_name="c", subcore_axis_name="s")

  @functools.partial(
      pl.kernel,
      out_type=jax.ShapeDtypeStruct((B, D), jnp.float32),
      mesh=mesh,
      scratch_types=[
          pltpu.VMEM((2, CH), jnp.float32),
          pltpu.VMEM((2, CH), jnp.int32),
          pltpu.VMEM((CH, D), jnp.float32),
          pltpu.VMEM((CH, D), jnp.float32),
          pltpu.SemaphoreType.DMA,
          pltpu.SemaphoreType.DMA,
          pltpu.SemaphoreType.DMA,
          pltpu.SemaphoreType.DMA,
          pltpu.SemaphoreType.DMA,
          pltpu.SemaphoreType.DMA,
      ],
      compiler_params=pltpu.CompilerParams(use_tc_tiling_on_sc=False),
      interpret=interpret,
  )
  def table_lookup(x_hbm, table_hbm, out_hbm,
                   x_b, idx_b, rows0, rows1,
                   sem_x0, sem_x1, sem_g0, sem_g1, sem_s0, sem_s1):
    wid = lax.axis_index("s") * NC + lax.axis_index("c")
    base = wid * PER_W
    sem_x = (sem_x0, sem_x1)
    sem_g = (sem_g0, sem_g1)
    sem_s = (sem_s0, sem_s1)
    rows_b = (rows0, rows1)

    def x_off(c):
      return base + lax.rem(c, NCHUNK) * CH

    def load_x(c, buf):
      pltpu.make_async_copy(
          x_hbm.at[pl.ds(x_off(c), CH)], x_b.at[buf], sem_x[buf]).start()

    def wait_x(buf):
      pltpu.make_async_copy(
          x_hbm.at[pl.ds(base, CH)], x_b.at[buf], sem_x[buf]).wait()

    def bucketize(buf):
      for i in range(CH // 16):
        xv = x_b[buf, pl.ds(i * 16, 16)]
        idx_b[buf, pl.ds(i * 16, 16)] = jnp.minimum(
            (xv * float(BINS)).astype(jnp.int32), BINS - 1)

    def fire_gathers(buf):
      cps = [
          pltpu.make_async_copy(
              table_hbm.at[idx_b.at[buf, pl.ds(g * G, G)]],
              rows_b[buf].at[pl.ds(g * G, G)],
              sem_g[buf],
          )
          for g in range(NG)
      ]
      for cp in cps:
        cp.start()

    def wait_gathers(buf):
      for g in range(NG):
        pltpu.make_async_copy(
            table_hbm.at[idx_b.at[buf, pl.ds(g * G, G)]],
            rows_b[buf].at[pl.ds(g * G, G)],
            sem_g[buf],
        ).wait()

    def start_store(c, buf):
      pltpu.make_async_copy(
          rows_b[buf], out_hbm.at[pl.ds(base + c * CH, CH)], sem_s[buf]).start()

    def wait_store(buf):
      pltpu.make_async_copy(
          rows_b[buf], out_hbm.at[pl.ds(base, CH)], sem_s[buf]).wait()

    load_x(0, 0)
    wait_x(0)
    bucketize(0)
    fire_gathers(0)
    load_x(1, 1)
    pltpu.make_async_copy(
        rows1, out_hbm.at[pl.ds(base + (NCHUNK - 1) * CH, CH)], sem_s1).start()

    def body(c, b, fire_next, prefetch=True):
      nb = 1 - b
      if fire_next:
        wait_x(nb)
        bucketize(nb)
        if prefetch:
          load_x(c + 2, b)
        wait_store(nb)
        fire_gathers(nb)
      wait_gathers(b)
      start_store(c, b)

    def pipe(t, carry):
      body(2 * t, 0, True)
      body(2 * t + 1, 1, True)
      return carry

    lax.fori_loop(0, NCHUNK // 2 - 1, pipe, 0)

    body(NCHUNK - 2, 0, True, prefetch=False)
    body(NCHUNK - 1, 1, False)

    wait_store(0)
    wait_store(1)

  return table_lookup


_lookup = _build()


def kernel(x, table):
  xf = x.reshape(B)
  out = _lookup(xf, table)
  return out.reshape(4096, 200, D)


# speedup vs baseline: 1.0215x; 1.0001x over previous
import functools

import jax
import jax.numpy as jnp
from jax import lax
from jax.experimental import pallas as pl
from jax.experimental.pallas import tpu as pltpu
from jax.experimental.pallas import tpu_sc as plsc

BINS = 1000000
D = 32
NC = 2
NS = 16
NW = NC * NS
B = 4096 * 200
PER_W = B // NW
CH = 512
G = 512
NG = CH // G
NCHUNK = PER_W // CH


def _build(interpret=False):
  mesh = plsc.VectorSubcoreMesh(core_axis_name="c", subcore_axis_name="s")

  @functools.partial(
      pl.kernel,
      out_type=jax.ShapeDtypeStruct((B, D), jnp.float32),
      mesh=mesh,
      scratch_types=[
          pltpu.VMEM((2, CH), jnp.float32),
          pltpu.VMEM((2, CH), jnp.int32),
          pltpu.VMEM((CH, D), jnp.float32),
          pltpu.VMEM((CH, D), jnp.float32),
          pltpu.SemaphoreType.DMA,
          pltpu.SemaphoreType.DMA,
          pltpu.SemaphoreType.DMA,
          pltpu.SemaphoreType.DMA,
          pltpu.SemaphoreType.DMA,
          pltpu.SemaphoreType.DMA,
      ],
      compiler_params=pltpu.CompilerParams(use_tc_tiling_on_sc=False),
      interpret=interpret,
  )
  def table_lookup(x_hbm, table_hbm, out_hbm,
                   x_b, idx_b, rows0, rows1,
                   sem_x0, sem_x1, sem_g0, sem_g1, sem_s0, sem_s1):
    wid = lax.axis_index("s") * NC + lax.axis_index("c")
    base = wid * PER_W
    sem_x = (sem_x0, sem_x1)
    sem_g = (sem_g0, sem_g1)
    sem_s = (sem_s0, sem_s1)
    rows_b = (rows0, rows1)

    def x_off(c):
      return base + lax.rem(c, NCHUNK) * CH

    def load_x(c, buf):
      pltpu.make_async_copy(
          x_hbm.at[pl.ds(x_off(c), CH)], x_b.at[buf], sem_x[buf]).start()

    def wait_x(buf):
      pltpu.make_async_copy(
          x_hbm.at[pl.ds(base, CH)], x_b.at[buf], sem_x[buf]).wait()

    def bucketize(buf):
      for i in range(CH // 16):
        xv = x_b[buf, pl.ds(i * 16, 16)]
        idx_b[buf, pl.ds(i * 16, 16)] = jnp.minimum(
            (xv * float(BINS)).astype(jnp.int32), BINS - 1)

    def fire_gathers(buf):
      cps = [
          pltpu.make_async_copy(
              table_hbm.at[idx_b.at[buf, pl.ds(g * G, G)]],
              rows_b[buf].at[pl.ds(g * G, G)],
              sem_g[buf],
          )
          for g in range(NG)
      ]
      for cp in cps:
        cp.start()

    def wait_gathers(buf):
      for g in range(NG):
        pltpu.make_async_copy(
            table_hbm.at[idx_b.at[buf, pl.ds(g * G, G)]],
            rows_b[buf].at[pl.ds(g * G, G)],
            sem_g[buf],
        ).wait()

    def start_store(c, buf):
      pltpu.make_async_copy(
          rows_b[buf], out_hbm.at[pl.ds(base + c * CH, CH)], sem_s[buf]).start()

    def wait_store(buf):
      pltpu.make_async_copy(
          rows_b[buf], out_hbm.at[pl.ds(base, CH)], sem_s[buf]).wait()

    load_x(0, 0)
    wait_x(0)
    bucketize(0)
    fire_gathers(0)
    load_x(1, 1)
    pltpu.make_async_copy(
        rows1, out_hbm.at[pl.ds(base + (NCHUNK - 1) * CH, CH)], sem_s1).start()

    def body(c, b, fire_next, prefetch=True):
      nb = 1 - b
      if fire_next:
        wait_x(nb)
        bucketize(nb)
        if prefetch:
          load_x(c + 2, b)
        wait_store(nb)
        fire_gathers(nb)
      wait_gathers(b)
      start_store(c, b)

    def pipe(t, carry):
      body(2 * t, 0, True)
      body(2 * t + 1, 1, True)
      return carry

    lax.fori_loop(0, NCHUNK // 2 - 1, pipe, 0)

    body(NCHUNK - 2, 0, True, prefetch=False)
    body(NCHUNK - 1, 1, False)

    wait_store(0)
    wait_store(1)

  return table_lookup


_lookup = _build()


def kernel(x, table):
  xf = x.reshape(B)
  out = _lookup(xf, table)
  return out.reshape(4096, 200, D)


# speedup vs baseline: 1.0223x; 1.0008x over previous
import functools

import jax
import jax.numpy as jnp
from jax import lax
from jax.experimental import pallas as pl
from jax.experimental.pallas import tpu as pltpu
from jax.experimental.pallas import tpu_sc as plsc

BINS = 1000000
D = 32
NC = 2
NS = 16
NW = NC * NS
B = 4096 * 200
PER_W = B // NW
CH = 512
G = 128
NG = CH // G
NCHUNK = PER_W // CH


def _build(interpret=False):
  mesh = plsc.VectorSubcoreMesh(core_axis_name="c", subcore_axis_name="s")

  @functools.partial(
      pl.kernel,
      out_type=jax.ShapeDtypeStruct((B, D), jnp.float32),
      mesh=mesh,
      scratch_types=[
          pltpu.VMEM((2, CH), jnp.float32),
          pltpu.VMEM((2, CH), jnp.int32),
          pltpu.VMEM((CH, D), jnp.float32),
          pltpu.VMEM((CH, D), jnp.float32),
          pltpu.SemaphoreType.DMA,
          pltpu.SemaphoreType.DMA,
          pltpu.SemaphoreType.DMA,
          pltpu.SemaphoreType.DMA,
          pltpu.SemaphoreType.DMA,
          pltpu.SemaphoreType.DMA,
      ],
      compiler_params=pltpu.CompilerParams(use_tc_tiling_on_sc=False),
      interpret=interpret,
  )
  def table_lookup(x_hbm, table_hbm, out_hbm,
                   x_b, idx_b, rows0, rows1,
                   sem_x0, sem_x1, sem_g0, sem_g1, sem_s0, sem_s1):
    wid = lax.axis_index("s") * NC + lax.axis_index("c")
    base = wid * PER_W
    sem_x = (sem_x0, sem_x1)
    sem_g = (sem_g0, sem_g1)
    sem_s = (sem_s0, sem_s1)
    rows_b = (rows0, rows1)

    def x_off(c):
      return base + lax.rem(c, NCHUNK) * CH

    def load_x(c, buf):
      pltpu.make_async_copy(
          x_hbm.at[pl.ds(x_off(c), CH)], x_b.at[buf], sem_x[buf]).start()

    def wait_x(buf):
      pltpu.make_async_copy(
          x_hbm.at[pl.ds(base, CH)], x_b.at[buf], sem_x[buf]).wait()

    def bucketize(buf):
      for i in range(CH // 16):
        xv = x_b[buf, pl.ds(i * 16, 16)]
        idx_b[buf, pl.ds(i * 16, 16)] = jnp.minimum(
            (xv * float(BINS)).astype(jnp.int32), BINS - 1)

    def fire_gathers(buf):
      cps = [
          pltpu.make_async_copy(
              table_hbm.at[idx_b.at[buf, pl.ds(g * G, G)]],
              rows_b[buf].at[pl.ds(g * G, G)],
              sem_g[buf],
          )
          for g in range(NG)
      ]
      for cp in cps:
        cp.start()

    def wait_gathers(buf):
      for g in range(NG):
        pltpu.make_async_copy(
            table_hbm.at[idx_b.at[buf, pl.ds(g * G, G)]],
            rows_b[buf].at[pl.ds(g * G, G)],
            sem_g[buf],
        ).wait()

    def start_store(c, buf):
      pltpu.make_async_copy(
          rows_b[buf], out_hbm.at[pl.ds(base + c * CH, CH)], sem_s[buf]).start()

    def wait_store(buf):
      pltpu.make_async_copy(
          rows_b[buf], out_hbm.at[pl.ds(base, CH)], sem_s[buf]).wait()

    load_x(0, 0)
    wait_x(0)
    bucketize(0)
    fire_gathers(0)
    load_x(1, 1)
    pltpu.make_async_copy(
        rows1, out_hbm.at[pl.ds(base + (NCHUNK - 1) * CH, CH)], sem_s1).start()

    def body(c, b, fire_next, prefetch=True):
      nb = 1 - b
      if fire_next:
        wait_x(nb)
        bucketize(nb)
        if prefetch:
          load_x(c + 2, b)
        wait_store(nb)
        fire_gathers(nb)
      wait_gathers(b)
      start_store(c, b)

    def pipe(t, carry):
      body(2 * t, 0, True)
      body(2 * t + 1, 1, True)
      return carry

    lax.fori_loop(0, NCHUNK // 2 - 1, pipe, 0)

    body(NCHUNK - 2, 0, True, prefetch=False)
    body(NCHUNK - 1, 1, False)

    wait_store(0)
    wait_store(1)

  return table_lookup


_lookup = _build()


def kernel(x, table):
  xf = x.reshape(B)
  out = _lookup(xf, table)
  return out.reshape(4096, 200, D)


# speedup vs baseline: 1.0227x; 1.0004x over previous
import functools

import jax
import jax.numpy as jnp
from jax import lax
from jax.experimental import pallas as pl
from jax.experimental.pallas import tpu as pltpu
from jax.experimental.pallas import tpu_sc as plsc

BINS = 1000000
D = 32
NC = 2
NS = 16
NW = NC * NS
B = 4096 * 200
PER_W = B // NW
CH = 512
G = 128
NG = CH // G
NCHUNK = PER_W // CH


def _build():
  mesh = plsc.VectorSubcoreMesh(core_axis_name="c", subcore_axis_name="s")

  @functools.partial(
      pl.kernel,
      out_type=jax.ShapeDtypeStruct((B, D), jnp.float32),
      mesh=mesh,
      scratch_types=[
          pltpu.VMEM((2, CH), jnp.float32),
          pltpu.VMEM((2, CH), jnp.int32),
          pltpu.VMEM((CH, D), jnp.float32),
          pltpu.VMEM((CH, D), jnp.float32),
          pltpu.SemaphoreType.DMA,
          pltpu.SemaphoreType.DMA,
          pltpu.SemaphoreType.DMA,
          pltpu.SemaphoreType.DMA,
          pltpu.SemaphoreType.DMA,
          pltpu.SemaphoreType.DMA,
      ],
      compiler_params=pltpu.CompilerParams(use_tc_tiling_on_sc=False),
  )
  def table_lookup(x_hbm, table_hbm, out_hbm,
                   x_b, idx_b, rows0, rows1,
                   sem_x0, sem_x1, sem_g0, sem_g1, sem_s0, sem_s1):
    wid = lax.axis_index("s") * NC + lax.axis_index("c")
    base = wid * PER_W
    sem_x = (sem_x0, sem_x1)
    sem_g = (sem_g0, sem_g1)
    sem_s = (sem_s0, sem_s1)
    rows_b = (rows0, rows1)

    def x_off(c):
      return base + lax.rem(c, NCHUNK) * CH

    def load_x(c, buf):
      pltpu.make_async_copy(
          x_hbm.at[pl.ds(x_off(c), CH)], x_b.at[buf], sem_x[buf]).start()

    def wait_x(buf):
      pltpu.make_async_copy(
          x_hbm.at[pl.ds(base, CH)], x_b.at[buf], sem_x[buf]).wait()

    def bucketize(buf):
      for i in range(CH // 16):
        xv = x_b[buf, pl.ds(i * 16, 16)]
        idx_b[buf, pl.ds(i * 16, 16)] = jnp.minimum(
            (xv * float(BINS)).astype(jnp.int32), BINS - 1)

    def fire_gathers(buf):
      cps = [
          pltpu.make_async_copy(
              table_hbm.at[idx_b.at[buf, pl.ds(g * G, G)]],
              rows_b[buf].at[pl.ds(g * G, G)],
              sem_g[buf],
          )
          for g in range(NG)
      ]
      for cp in cps:
        cp.start()

    def wait_gathers(buf):
      for g in range(NG):
        pltpu.make_async_copy(
            table_hbm.at[idx_b.at[buf, pl.ds(g * G, G)]],
            rows_b[buf].at[pl.ds(g * G, G)],
            sem_g[buf],
        ).wait()

    def start_store(c, buf):
      pltpu.make_async_copy(
          rows_b[buf], out_hbm.at[pl.ds(base + c * CH, CH)], sem_s[buf]).start()

    def wait_store(buf):
      pltpu.make_async_copy(
          rows_b[buf], out_hbm.at[pl.ds(base, CH)], sem_s[buf]).wait()

    load_x(0, 0)
    wait_x(0)
    bucketize(0)
    fire_gathers(0)
    load_x(1, 1)
    pltpu.make_async_copy(
        rows1, out_hbm.at[pl.ds(base + (NCHUNK - 1) * CH, CH)], sem_s1).start()

    def body(c, b, fire_next, prefetch=True):
      nb = 1 - b
      if fire_next:
        wait_x(nb)
        bucketize(nb)
        if prefetch:
          load_x(c + 2, b)
        wait_store(nb)
        fire_gathers(nb)
      wait_gathers(b)
      start_store(c, b)

    def pipe(t, carry):
      body(2 * t, 0, True)
      body(2 * t + 1, 1, True)
      return carry

    lax.fori_loop(0, NCHUNK // 2 - 1, pipe, 0)

    body(NCHUNK - 2, 0, True, prefetch=False)
    body(NCHUNK - 1, 1, False)

    wait_store(0)
    wait_store(1)

  return table_lookup


_lookup = _build()


def kernel(x, table):
  xf = x.reshape(B)
  out = _lookup(xf, table)
  return out.reshape(4096, 200, D)
